# Initial kernel scaffold; baseline (speedup 1.0000x reference)
import jax
import jax.numpy as jnp
from jax.experimental import pallas as pl


def kernel(x, edge_index, W1, b1, W2, b2, W3, b3):
    raise NotImplementedError("write your pallas kernel here")


# speedup vs baseline: 7.7984x; 7.7984x over previous
import functools

import jax
import jax.numpy as jnp
from jax import lax
from jax.experimental import pallas as pl
from jax.experimental.pallas import tpu as pltpu
from jax.experimental.pallas import tpu_sc as plsc

N_NODES = 10000
D = 128
K = 128
NBUF = 2
NC = 2
NS = 16
NW = NC * NS
CHUNKS = 80
N_ACC = 10112
ZROWS = N_ACC // NS

_mesh = plsc.VectorSubcoreMesh(core_axis_name="c", subcore_axis_name="s")


@functools.partial(
    pl.kernel,
    mesh=_mesh,
    out_type=jax.ShapeDtypeStruct((NC, N_ACC, D), jnp.float32),
    scratch_types=[
        pltpu.VMEM_SHARED((N_ACC, D), jnp.float32),
        pltpu.VMEM((NBUF, K), jnp.int32),
        pltpu.VMEM((K, D), jnp.float32),
    ],
)
def _deg_kernel(dst_hbm, ones_hbm, zeros_hbm, out_hbm, accd, db, ones_v):
    c = lax.axis_index("c")
    s = lax.axis_index("s")
    w = c * NS + s
    pltpu.sync_copy(zeros_hbm.at[pl.ds(s * ZROWS, ZROWS)],
                    accd.at[pl.ds(s * ZROWS, ZROWS)])
    pltpu.sync_copy(ones_hbm, ones_v)
    plsc.subcore_barrier()

    def body(i, carry):
        pltpu.sync_copy(dst_hbm.at[w, i], db.at[0])
        pltpu.sync_copy(ones_v, accd.at[db.at[0]], add=True)
        return carry

    lax.fori_loop(0, CHUNKS, body, 0)
    plsc.subcore_barrier()
    pltpu.sync_copy(accd.at[pl.ds(s * ZROWS, ZROWS)],
                    out_hbm.at[c, pl.ds(s * ZROWS, ZROWS)])


@functools.partial(
    pl.kernel,
    mesh=_mesh,
    out_type=jax.ShapeDtypeStruct((NC, N_ACC, D), jnp.float32),
    scratch_types=[
        pltpu.VMEM_SHARED((N_ACC, D), jnp.float32),
        pltpu.VMEM((NBUF, K), jnp.int32),
        pltpu.VMEM((NBUF, K), jnp.int32),
        pltpu.VMEM((NBUF, K, D), jnp.float32),
        pltpu.SemaphoreType.DMA,
        pltpu.SemaphoreType.DMA,
    ],
)
def _agg_kernel(y_hbm, src_hbm, dst_hbm, zeros_hbm, out_hbm,
                acc, sb, db, rows, sem0, sem1):
    c = lax.axis_index("c")
    s = lax.axis_index("s")
    w = c * NS + s
    pltpu.sync_copy(zeros_hbm.at[pl.ds(s * ZROWS, ZROWS)],
                    acc.at[pl.ds(s * ZROWS, ZROWS)])
    plsc.subcore_barrier()

    sems = [sem0, sem1]
    for b in range(NBUF):
        pltpu.sync_copy(src_hbm.at[w, b], sb.at[b])
        pltpu.sync_copy(dst_hbm.at[w, b], db.at[b])
        pltpu.make_async_copy(y_hbm.at[sb.at[b]], rows.at[b], sems[b]).start()

    def body(it, carry):
        g = it * NBUF
        for b in range(NBUF):
            i = g + b
            pltpu.make_async_copy(y_hbm.at[sb.at[b]], rows.at[b],
                                  sems[b]).wait()
            pltpu.sync_copy(rows.at[b], acc.at[db.at[b]], add=True)
            j = i + NBUF

            @pl.when(j < CHUNKS)
            def _prefetch():
                pltpu.sync_copy(src_hbm.at[w, j], sb.at[b])
                pltpu.sync_copy(dst_hbm.at[w, j], db.at[b])
                pltpu.make_async_copy(y_hbm.at[sb.at[b]], rows.at[b],
                                      sems[b]).start()

        return carry

    lax.fori_loop(0, CHUNKS // NBUF, body, 0)
    plsc.subcore_barrier()
    pltpu.sync_copy(acc.at[pl.ds(s * ZROWS, ZROWS)],
                    out_hbm.at[c, pl.ds(s * ZROWS, ZROWS)])


_BLK = 1000
_GRID = N_NODES // _BLK


def _dis_from(deg0_ref, deg1_ref):
    return lax.rsqrt(1.0 + deg0_ref[...][:, 0:1] + deg1_ref[...][:, 0:1])


def _first_body(deg0_ref, deg1_ref, x_ref, w_ref, y_ref):
    dis = _dis_from(deg0_ref, deg1_ref)
    y_ref[...] = dis * jnp.dot(x_ref[...], w_ref[...],
                               preferred_element_type=jnp.float32)


def _mid_body(p0_ref, p1_ref, y_ref, deg0_ref, deg1_ref, b_ref, w_ref,
              out_ref):
    dis = _dis_from(deg0_ref, deg1_ref)
    h = dis * (p0_ref[...] + p1_ref[...] + y_ref[...]) + b_ref[...]
    h = jnp.maximum(h, 0.0)
    out_ref[...] = dis * jnp.dot(h, w_ref[...],
                                 preferred_element_type=jnp.float32)


def _last_body(p0_ref, p1_ref, y_ref, deg0_ref, deg1_ref, b_ref, out_ref):
    dis = _dis_from(deg0_ref, deg1_ref)
    out_ref[...] = dis * (p0_ref[...] + p1_ref[...] + y_ref[...]) + b_ref[...]


_row_spec = pl.BlockSpec((_BLK, D), lambda i: (i, 0))
_deg_spec = _row_spec
_w_spec = pl.BlockSpec((D, D), lambda i: (0, 0))
_b_spec = pl.BlockSpec((1, D), lambda i: (0, 0))
_out_row = jax.ShapeDtypeStruct((N_NODES, D), jnp.float32)

_first_call = pl.pallas_call(
    _first_body, grid=(_GRID,),
    in_specs=[_deg_spec, _deg_spec, _row_spec, _w_spec],
    out_specs=_row_spec, out_shape=_out_row)

_mid_call = pl.pallas_call(
    _mid_body, grid=(_GRID,),
    in_specs=[_row_spec, _row_spec, _row_spec, _deg_spec, _deg_spec,
              _b_spec, _w_spec],
    out_specs=_row_spec, out_shape=_out_row)

_last_call = pl.pallas_call(
    _last_body, grid=(_GRID,),
    in_specs=[_row_spec, _row_spec, _row_spec, _deg_spec, _deg_spec, _b_spec],
    out_specs=_row_spec, out_shape=_out_row)


def kernel(x, edge_index, W1, b1, W2, b2, W3, b3):
    e = edge_index.shape[1]
    e_pad = NW * CHUNKS * K
    pad = e_pad - e
    src = jnp.concatenate(
        [edge_index[0], jnp.zeros((pad,), jnp.int32)]).reshape(NW, CHUNKS, K)
    dst = jnp.concatenate(
        [edge_index[1],
         jnp.full((pad,), N_NODES, jnp.int32)]).reshape(NW, CHUNKS, K)

    zeros128 = jnp.zeros((N_ACC, D), jnp.float32)
    ones128 = jnp.ones((K, D), jnp.float32)
    b1r = b1.reshape(1, D)
    b2r = b2.reshape(1, D)
    b3r = b3.reshape(1, D)

    deg2 = _deg_kernel(dst, ones128, zeros128)
    deg0, deg1 = deg2[0], deg2[1]

    y1 = _first_call(deg0, deg1, x, W1)
    p1 = _agg_kernel(y1, src, dst, zeros128)
    y2 = _mid_call(p1[0], p1[1], y1, deg0, deg1, b1r, W2)
    p2 = _agg_kernel(y2, src, dst, zeros128)
    y3 = _mid_call(p2[0], p2[1], y2, deg0, deg1, b2r, W3)
    p3 = _agg_kernel(y3, src, dst, zeros128)
    return _last_call(p3[0], p3[1], y3, deg0, deg1, b3r)
